# baseline (speedup 1.0000x reference)
import functools

import jax
import jax.numpy as jnp
from jax import lax
from jax.experimental import pallas as pl
from jax.experimental.pallas import tpu as pltpu
from jax.experimental.pallas import tpu_sc as plsc

_CHUNK = 128


@functools.partial(jax.jit, static_argnames=("n_workers",))
def _run(idx_flat, partidx, weight_groups, n_workers):
    n = idx_flat.shape[0]
    d = weight_groups.shape[1]
    per_w = n // n_workers
    n_chunks = per_w // _CHUNK

    mesh = plsc.VectorSubcoreMesh(core_axis_name="c", subcore_axis_name="s")

    @functools.partial(
        pl.kernel,
        mesh=mesh,
        compiler_params=pltpu.CompilerParams(use_tc_tiling_on_sc=False),
        out_type=jax.ShapeDtypeStruct((n, d), jnp.float32),
        scratch_types=[
            pltpu.VMEM((_CHUNK,), jnp.int32),
            pltpu.VMEM((_CHUNK,), jnp.int32),
            pltpu.VMEM((_CHUNK, d), jnp.float32),
            pltpu.SemaphoreType.DMA,
        ],
    )
    def sc_kernel(idx_hbm, part_hbm, wg_hbm, out_hbm, idx_v, p_v, rows_v, sem):
        wid = lax.axis_index("s") * 2 + lax.axis_index("c")
        base = wid * per_w

        def body(j, carry):
            off = base + j * _CHUNK
            pltpu.sync_copy(idx_hbm.at[pl.ds(off, _CHUNK)], idx_v)
            pltpu.async_copy(part_hbm.at[idx_v], p_v, sem).wait()
            pltpu.async_copy(wg_hbm.at[p_v], rows_v, sem).wait()
            pltpu.sync_copy(rows_v, out_hbm.at[pl.ds(off, _CHUNK)])
            return carry

        lax.fori_loop(0, n_chunks, body, 0)

    return sc_kernel(idx_flat, partidx, weight_groups)


def kernel(input_idx, decoder, partidx, weight_groups):
    b, l = input_idx.shape
    d = weight_groups.shape[1]
    idx_flat = input_idx.reshape(b * l)
    out = _run(idx_flat, partidx, weight_groups, 32)
    return out.reshape(b, l, d)


# speedup vs baseline: 1.0005x; 1.0005x over previous
import functools

import jax
import jax.numpy as jnp
from jax import lax
from jax.experimental import pallas as pl
from jax.experimental.pallas import tpu as pltpu
from jax.experimental.pallas import tpu_sc as plsc

_C = 128
_CPB = 4
_BLK = _C * _CPB


@functools.partial(jax.jit, static_argnames=("n_workers",))
def _run(idx2d, partidx, weight_groups, n_workers):
    n_rows, c = idx2d.shape
    n = n_rows * c
    d = weight_groups.shape[1]
    per_w = n // n_workers
    chunks_w = per_w // _C
    blocks_w = chunks_w // _CPB
    sblocks_w = blocks_w // 2

    mesh = plsc.VectorSubcoreMesh(core_axis_name="c", subcore_axis_name="s")

    @functools.partial(
        pl.kernel,
        mesh=mesh,
        compiler_params=pltpu.CompilerParams(use_tc_tiling_on_sc=False),
        out_type=jax.ShapeDtypeStruct((n, d), jnp.float32),
        scratch_types=[
            pltpu.VMEM((chunks_w, _C), jnp.int32),
            pltpu.VMEM((chunks_w, _C), jnp.int32),
            pltpu.VMEM((2, _BLK, d), jnp.float32),
            pltpu.SemaphoreType.DMA,
            pltpu.SemaphoreType.DMA,
            pltpu.SemaphoreType.DMA,
            pltpu.SemaphoreType.DMA,
        ],
    )
    def sc_kernel(idx_hbm, part_hbm, wg_hbm, out_hbm,
                  idx_v, p_v, rows_v, semP, semR, semO0, semO1):
        wid = lax.axis_index("s") * 2 + lax.axis_index("c")
        row_base = wid * chunks_w
        base = wid * per_w

        pltpu.sync_copy(idx_hbm.at[pl.ds(row_base, chunks_w)], idx_v)

        def fire_p(j, carry):
            pltpu.async_copy(part_hbm.at[idx_v.at[j]], p_v.at[j], semP)
            return carry
        lax.fori_loop(0, chunks_w, fire_p, 0)
        pltpu.make_async_copy(idx_hbm.at[pl.ds(row_base, chunks_w)], p_v,
                              semP).wait()

        def do_block(b, slot_sem):
            s, sem = slot_sem
            for t in range(_CPB):
                j = b * _CPB + t
                pltpu.async_copy(wg_hbm.at[p_v.at[j]],
                                 rows_v.at[s, pl.ds(t * _C, _C)], semR)
            off = base + b * _BLK
            pltpu.make_async_copy(out_hbm.at[pl.ds(off, _BLK)], rows_v.at[s],
                                  semR).wait()
            pltpu.async_copy(rows_v.at[s], out_hbm.at[pl.ds(off, _BLK)], sem)

        def sblock(t, carry):
            @pl.when(t > 0)
            def _():
                pltpu.make_async_copy(
                    rows_v.at[0], out_hbm.at[pl.ds(base, _BLK)], semO0).wait()
            do_block(2 * t, (0, semO0))

            @pl.when(t > 0)
            def _():
                pltpu.make_async_copy(
                    rows_v.at[1], out_hbm.at[pl.ds(base, _BLK)], semO1).wait()
            do_block(2 * t + 1, (1, semO1))
            return carry
        lax.fori_loop(0, sblocks_w, sblock, 0)

        pltpu.make_async_copy(
            rows_v.at[0], out_hbm.at[pl.ds(base, _BLK)], semO0).wait()
        pltpu.make_async_copy(
            rows_v.at[1], out_hbm.at[pl.ds(base, _BLK)], semO1).wait()

    return sc_kernel(idx2d, partidx, weight_groups)


def kernel(input_idx, decoder, partidx, weight_groups):
    b, l = input_idx.shape
    d = weight_groups.shape[1]
    n = b * l
    idx2d = input_idx.reshape(n // _C, _C)
    out = _run(idx2d, partidx, weight_groups, 32)
    return out.reshape(b, l, d)


# speedup vs baseline: 5.5319x; 5.5293x over previous
import functools

import jax
import jax.numpy as jnp
from jax import lax
from jax.experimental import pallas as pl
from jax.experimental.pallas import tpu as pltpu
from jax.experimental.pallas import tpu_sc as plsc

_C = 128
_CPB = 4
_BLK = _C * _CPB


@functools.partial(jax.jit, static_argnames=("n_workers",))
def _run(idx2d, partidx, weight_groups, n_workers):
    n_rows, c = idx2d.shape
    n = n_rows * c
    p, d = weight_groups.shape
    per_w = n // n_workers
    chunks_w = per_w // _C
    blocks_w = chunks_w // _CPB
    sblocks_w = blocks_w // 2

    mesh = plsc.VectorSubcoreMesh(core_axis_name="c", subcore_axis_name="s")

    @functools.partial(
        pl.kernel,
        mesh=mesh,
        compiler_params=pltpu.CompilerParams(use_tc_tiling_on_sc=False),
        out_type=jax.ShapeDtypeStruct((n, d), jnp.float32),
        scratch_types=[
            pltpu.VMEM((chunks_w, _C), jnp.int32),
            pltpu.VMEM((chunks_w, _C), jnp.int32),
            pltpu.VMEM((2, _BLK, d), jnp.float32),
            pltpu.VMEM_SHARED((p, d), jnp.float32),
            pltpu.SemaphoreType.DMA,
            pltpu.SemaphoreType.DMA,
            pltpu.SemaphoreType.DMA,
            pltpu.SemaphoreType.DMA,
        ],
    )
    def sc_kernel(idx_hbm, part_hbm, wg_hbm, out_hbm,
                  idx_v, p_v, rows_v, wg_sh, semP, semR, semO0, semO1):
        wid = lax.axis_index("s") * 2 + lax.axis_index("c")
        row_base = wid * chunks_w
        base = wid * per_w

        @pl.when(lax.axis_index("s") == 0)
        def _():
            pltpu.sync_copy(wg_hbm, wg_sh)
        plsc.subcore_barrier()

        pltpu.sync_copy(idx_hbm.at[pl.ds(row_base, chunks_w)], idx_v)

        def fire_p(j, carry):
            pltpu.async_copy(part_hbm.at[idx_v.at[j]], p_v.at[j], semP)
            return carry
        lax.fori_loop(0, chunks_w, fire_p, 0)
        pltpu.make_async_copy(idx_hbm.at[pl.ds(row_base, chunks_w)], p_v,
                              semP).wait()

        def do_block(b, slot_sem):
            s, sem = slot_sem
            for t in range(_CPB):
                j = b * _CPB + t
                pltpu.async_copy(wg_sh.at[p_v.at[j]],
                                 rows_v.at[s, pl.ds(t * _C, _C)], semR)
            off = base + b * _BLK
            pltpu.make_async_copy(out_hbm.at[pl.ds(off, _BLK)], rows_v.at[s],
                                  semR).wait()
            pltpu.async_copy(rows_v.at[s], out_hbm.at[pl.ds(off, _BLK)], sem)

        def sblock(t, carry):
            @pl.when(t > 0)
            def _():
                pltpu.make_async_copy(
                    rows_v.at[0], out_hbm.at[pl.ds(base, _BLK)], semO0).wait()
            do_block(2 * t, (0, semO0))

            @pl.when(t > 0)
            def _():
                pltpu.make_async_copy(
                    rows_v.at[1], out_hbm.at[pl.ds(base, _BLK)], semO1).wait()
            do_block(2 * t + 1, (1, semO1))
            return carry
        lax.fori_loop(0, sblocks_w, sblock, 0)

        pltpu.make_async_copy(
            rows_v.at[0], out_hbm.at[pl.ds(base, _BLK)], semO0).wait()
        pltpu.make_async_copy(
            rows_v.at[1], out_hbm.at[pl.ds(base, _BLK)], semO1).wait()

    return sc_kernel(idx2d, partidx, weight_groups)


def kernel(input_idx, decoder, partidx, weight_groups):
    b, l = input_idx.shape
    d = weight_groups.shape[1]
    n = b * l
    idx2d = input_idx.reshape(n // _C, _C)
    out = _run(idx2d, partidx, weight_groups, 32)
    return out.reshape(b, l, d)


# speedup vs baseline: 5.5727x; 1.0074x over previous
import functools

import jax
import jax.numpy as jnp
from jax import lax
from jax.experimental import pallas as pl
from jax.experimental.pallas import tpu as pltpu
from jax.experimental.pallas import tpu_sc as plsc

_C = 128
_CPB = 2
_BLK = _C * _CPB


@functools.partial(jax.jit, static_argnames=("n_workers",))
def _run(idx2d, partidx, weight_groups, n_workers):
    n_rows, c = idx2d.shape
    n = n_rows * c
    p, d = weight_groups.shape
    n_tok = partidx.shape[0]
    n_stage = 4
    stage_chunk = n_tok // n_stage
    assert stage_chunk % 8 == 0 and stage_chunk * n_stage == n_tok
    per_w = n // n_workers
    chunks_w = per_w // _C
    blocks_w = chunks_w // _CPB
    sblocks_w = blocks_w // 2

    mesh = plsc.VectorSubcoreMesh(core_axis_name="c", subcore_axis_name="s")

    @functools.partial(
        pl.kernel,
        mesh=mesh,
        compiler_params=pltpu.CompilerParams(use_tc_tiling_on_sc=False),
        out_type=jax.ShapeDtypeStruct((n, d), jnp.float32),
        scratch_types=[
            pltpu.VMEM((chunks_w, _C), jnp.int32),
            pltpu.VMEM((chunks_w, _C), jnp.int32),
            pltpu.VMEM((2, _BLK, d), jnp.float32),
            pltpu.VMEM_SHARED((p, d), jnp.float32),
            pltpu.VMEM_SHARED((n_tok,), jnp.int32),
            pltpu.SemaphoreType.DMA,
            pltpu.SemaphoreType.DMA,
            pltpu.SemaphoreType.DMA,
            pltpu.SemaphoreType.DMA,
        ],
    )
    def sc_kernel(idx_hbm, part_hbm, wg_hbm, out_hbm,
                  idx_v, p_v, rows_v, wg_sh, part_sh, semP, semR, semO0, semO1):
        sid = lax.axis_index("s")
        wid = sid * 2 + lax.axis_index("c")
        row_base = wid * chunks_w
        base = wid * per_w

        @pl.when(sid == 0)
        def _():
            pltpu.sync_copy(wg_hbm, wg_sh)

        @pl.when(sid < n_stage)
        def _():
            pltpu.sync_copy(part_hbm.at[pl.ds(sid * stage_chunk, stage_chunk)],
                            part_sh.at[pl.ds(sid * stage_chunk, stage_chunk)])
        plsc.subcore_barrier()

        pltpu.sync_copy(idx_hbm.at[pl.ds(row_base, chunks_w)], idx_v)

        def fire_p(j, carry):
            pltpu.async_copy(part_sh.at[idx_v.at[j]], p_v.at[j], semP)
            return carry
        lax.fori_loop(0, chunks_w, fire_p, 0)
        pltpu.make_async_copy(idx_hbm.at[pl.ds(row_base, chunks_w)], p_v,
                              semP).wait()

        def do_block(b, slot_sem):
            s, sem = slot_sem
            for t in range(_CPB):
                j = b * _CPB + t
                pltpu.async_copy(wg_sh.at[p_v.at[j]],
                                 rows_v.at[s, pl.ds(t * _C, _C)], semR)
            off = base + b * _BLK
            pltpu.make_async_copy(out_hbm.at[pl.ds(off, _BLK)], rows_v.at[s],
                                  semR).wait()
            pltpu.async_copy(rows_v.at[s], out_hbm.at[pl.ds(off, _BLK)], sem)

        def sblock(t, carry):
            @pl.when(t > 0)
            def _():
                pltpu.make_async_copy(
                    rows_v.at[0], out_hbm.at[pl.ds(base, _BLK)], semO0).wait()
            do_block(2 * t, (0, semO0))

            @pl.when(t > 0)
            def _():
                pltpu.make_async_copy(
                    rows_v.at[1], out_hbm.at[pl.ds(base, _BLK)], semO1).wait()
            do_block(2 * t + 1, (1, semO1))
            return carry
        lax.fori_loop(0, sblocks_w, sblock, 0)

        pltpu.make_async_copy(
            rows_v.at[0], out_hbm.at[pl.ds(base, _BLK)], semO0).wait()
        pltpu.make_async_copy(
            rows_v.at[1], out_hbm.at[pl.ds(base, _BLK)], semO1).wait()

    return sc_kernel(idx2d, partidx, weight_groups)


def kernel(input_idx, decoder, partidx, weight_groups):
    b, l = input_idx.shape
    d = weight_groups.shape[1]
    n = b * l
    idx2d = input_idx.reshape(n // _C, _C)
    out = _run(idx2d, partidx, weight_groups, 32)
    return out.reshape(b, l, d)
